# Initial kernel scaffold; baseline (speedup 1.0000x reference)
import jax
import jax.numpy as jnp
from jax.experimental import pallas as pl


def kernel(x, W_gating, gating_bias, W_heads, b_heads):
    raise NotImplementedError("write your pallas kernel here")


# speedup vs baseline: 8.4781x; 8.4781x over previous
import jax
import jax.numpy as jnp
from jax.experimental import pallas as pl

B = 4096
D = 768
NUM_TASKS = 8
HEAD_DIM = 64
BLK = 512


def _heads_kernel(x_ref, w_ref, b_ref, out_ref):
    xb = x_ref[...]
    feats = xb[:, :D]
    task = xb[:, D:]
    t = jnp.argmax(task, axis=-1)
    fo = jnp.tanh(feats)
    H = jnp.dot(fo, w_ref[...], preferred_element_type=jnp.float32)
    H = H + b_ref[...]
    acc = jnp.zeros((xb.shape[0], HEAD_DIM), jnp.float32)
    for tt in range(NUM_TASKS):
        sel = (t == tt)[:, None]
        acc = acc + jnp.where(sel, H[:, tt * HEAD_DIM:(tt + 1) * HEAD_DIM], 0.0)
    out_ref[...] = acc


def kernel(x, W_gating, gating_bias, W_heads, b_heads):
    W2d = W_heads.transpose(1, 0, 2).reshape(D, NUM_TASKS * HEAD_DIM)
    b2d = b_heads.reshape(1, NUM_TASKS * HEAD_DIM)
    grid = (B // BLK,)
    return pl.pallas_call(
        _heads_kernel,
        grid=grid,
        in_specs=[
            pl.BlockSpec((BLK, D + NUM_TASKS), lambda i: (i, 0)),
            pl.BlockSpec((D, NUM_TASKS * HEAD_DIM), lambda i: (0, 0)),
            pl.BlockSpec((1, NUM_TASKS * HEAD_DIM), lambda i: (0, 0)),
        ],
        out_specs=pl.BlockSpec((BLK, HEAD_DIM), lambda i: (i, 0)),
        out_shape=jax.ShapeDtypeStruct((B, HEAD_DIM), jnp.float32),
    )(x, W2d, b2d)


# speedup vs baseline: 8.7861x; 1.0363x over previous
import jax
import jax.numpy as jnp
from jax.experimental import pallas as pl
from jax.experimental.pallas import tpu as pltpu

B = 4096
D = 768
NUM_TASKS = 8
HEAD_DIM = 64
BLK = 1024


def _heads_kernel(x_ref, w_ref, b_ref, out_ref, w2d_ref):
    @pl.when(pl.program_id(0) == 0)
    def _build_w2d():
        for tt in range(NUM_TASKS):
            w2d_ref[:, tt * HEAD_DIM:(tt + 1) * HEAD_DIM] = w_ref[tt]

    xb = x_ref[...]
    feats = xb[:, :D]
    task = xb[:, D:]
    t = jnp.argmax(task, axis=-1)
    fo = jnp.tanh(feats)
    H = jnp.dot(fo, w2d_ref[...], preferred_element_type=jnp.float32)
    H = H + b_ref[...]
    acc = jnp.zeros((xb.shape[0], HEAD_DIM), jnp.float32)
    for tt in range(NUM_TASKS):
        sel = (t == tt)[:, None]
        acc = acc + jnp.where(sel, H[:, tt * HEAD_DIM:(tt + 1) * HEAD_DIM], 0.0)
    out_ref[...] = acc


def kernel(x, W_gating, gating_bias, W_heads, b_heads):
    b2d = b_heads.reshape(1, NUM_TASKS * HEAD_DIM)
    grid = (B // BLK,)
    return pl.pallas_call(
        _heads_kernel,
        grid=grid,
        in_specs=[
            pl.BlockSpec((BLK, D + NUM_TASKS), lambda i: (i, 0)),
            pl.BlockSpec((NUM_TASKS, D, HEAD_DIM), lambda i: (0, 0, 0)),
            pl.BlockSpec((1, NUM_TASKS * HEAD_DIM), lambda i: (0, 0)),
        ],
        out_specs=pl.BlockSpec((BLK, HEAD_DIM), lambda i: (i, 0)),
        out_shape=jax.ShapeDtypeStruct((B, HEAD_DIM), jnp.float32),
        scratch_shapes=[pltpu.VMEM((D, NUM_TASKS * HEAD_DIM), jnp.float32)],
    )(x, W_heads, b2d)
